# Initial kernel scaffold; baseline (speedup 1.0000x reference)
import jax
import jax.numpy as jnp
from jax.experimental import pallas as pl


def kernel(pc1, pc2, feature1, feature2, params):
    raise NotImplementedError("write your pallas kernel here")


# speedup vs baseline: 5.3508x; 5.3508x over previous
import functools

import jax
import jax.numpy as jnp
from jax import lax
from jax.experimental import pallas as pl
from jax.experimental.pallas import tpu as pltpu
from jax.experimental.pallas import tpu_sc as plsc

_BN = 1.0 / (1.0 + 1e-5) ** 0.5
_NW = 32


def _dotg(a, b_oc):
    return lax.dot_general(a, b_oc, (((1,), (1,)), ((), ())),
                           preferred_element_type=jnp.float32)


def _pick_ts(r, cap=1024):
    for t in (cap, 512, 256, 128, 64, 32, 16, 8):
        if t <= cap and r % t == 0 and t <= r:
            return t
    return r


def _fps_kernel(x_ref, o_ref, *, npoint, n):
    b = pl.program_id(0)
    x = x_ref[0]
    lane = lax.broadcasted_iota(jnp.int32, (1, n), 1)
    olane = lax.broadcasted_iota(jnp.int32, (1, npoint), 1)

    def body(i, st):
        dist, far, out = st
        out = jnp.where(olane == i, far, out)
        c = jnp.sum(jnp.where(lane == far, x, 0.0), axis=1, keepdims=True)
        d = jnp.sum((x - c) ** 2, axis=0, keepdims=True)
        dist = jnp.minimum(dist, d)
        m = jnp.max(dist)
        far = jnp.min(jnp.where(dist == m, lane, n))
        return dist, far, out

    dist0 = jnp.full((1, n), 1e10, jnp.float32)
    out0 = jnp.zeros((1, npoint), jnp.int32)
    _, _, out = lax.fori_loop(0, npoint, body, (dist0, jnp.int32(0), out0))
    o_ref[0] = out + b * n


def _fps(xt, npoint):
    b_, _, n = xt.shape
    out = pl.pallas_call(
        functools.partial(_fps_kernel, npoint=npoint, n=n),
        grid=(b_,),
        in_specs=[pl.BlockSpec((1, 16, n), lambda i: (i, 0, 0))],
        out_specs=pl.BlockSpec((1, 1, npoint), lambda i: (i, 0, 0)),
        out_shape=jax.ShapeDtypeStruct((b_, 1, npoint), jnp.int32),
    )(xt)
    return out.reshape(b_ * npoint)


def _sqd(q, x):
    qq = jnp.sum(q * q, axis=1, keepdims=True)
    xx = jnp.sum(x * x, axis=1, keepdims=True)
    dot = lax.dot_general(q, x, (((1,), (1,)), ((), ())),
                          preferred_element_type=jnp.float32)
    return qq + jnp.transpose(xx) - 2.0 * dot


def _ball_kernel(q_ref, x_ref, o_ref, *, k, r2, n):
    b = pl.program_id(0)
    d = _sqd(q_ref[0], x_ref[0])
    lane = lax.broadcasted_iota(jnp.int32, d.shape, 1)
    midx = jnp.where(d <= r2, lane, n)
    cols = []
    for _ in range(k):
        cur = jnp.min(midx, axis=1, keepdims=True)
        cols.append(cur)
        midx = jnp.where(midx == cur, n, midx)
    out = jnp.concatenate(cols, axis=1)
    out = jnp.where(out == n, cols[0], out)
    out = jnp.minimum(out, n - 1)
    o_ref[0] = out + b * n


def _ball(q_rows, x_rows, radius, k):
    b_, s, _ = q_rows.shape
    n = x_rows.shape[1]
    ts = _pick_ts(s, 256)
    out = pl.pallas_call(
        functools.partial(_ball_kernel, k=k, r2=radius * radius, n=n),
        grid=(b_, s // ts),
        in_specs=[
            pl.BlockSpec((1, ts, 16), lambda i, j: (i, j, 0)),
            pl.BlockSpec((1, n, 16), lambda i, j: (i, 0, 0)),
        ],
        out_specs=pl.BlockSpec((1, ts, k), lambda i, j: (i, j, 0)),
        out_shape=jax.ShapeDtypeStruct((b_, s, k), jnp.int32),
    )(q_rows, x_rows)
    return out


def _knn_kernel(q_ref, x_ref, *o_refs, k, n, want_d):
    b = pl.program_id(0)
    d = _sqd(q_ref[0], x_ref[0])
    lane = lax.broadcasted_iota(jnp.int32, d.shape, 1)
    icols, dcols = [], []
    for _ in range(k):
        m = jnp.min(d, axis=1, keepdims=True)
        j = jnp.min(jnp.where(d == m, lane, n), axis=1, keepdims=True)
        icols.append(j)
        dcols.append(m)
        d = jnp.where(lane == j, jnp.inf, d)
    o_refs[0][0] = jnp.concatenate(icols, axis=1) + b * n
    if want_d:
        o_refs[1][0] = jnp.concatenate(dcols, axis=1)


def _knn(q_rows, x_rows, k, want_d=False):
    b_, s, _ = q_rows.shape
    n = x_rows.shape[1]
    ts = _pick_ts(s, 256)
    shapes = [jax.ShapeDtypeStruct((b_, s, k), jnp.int32)]
    ospecs = [pl.BlockSpec((1, ts, k), lambda i, j: (i, j, 0))]
    if want_d:
        shapes.append(jax.ShapeDtypeStruct((b_, s, k), jnp.float32))
        ospecs.append(pl.BlockSpec((1, ts, k), lambda i, j: (i, j, 0)))
    out = pl.pallas_call(
        functools.partial(_knn_kernel, k=k, n=n, want_d=want_d),
        grid=(b_, s // ts),
        in_specs=[
            pl.BlockSpec((1, ts, 16), lambda i, j: (i, j, 0)),
            pl.BlockSpec((1, n, 16), lambda i, j: (i, 0, 0)),
        ],
        out_specs=ospecs,
        out_shape=shapes,
    )(q_rows, x_rows)
    return out if want_d else (out[0], None)


def _gather_rows(table, idx):
    r, = idx.shape
    d = table.shape[1]
    rp = -(-r // 256) * 256
    if rp != r:
        idx = jnp.concatenate([idx, jnp.zeros((rp - r,), jnp.int32)])
    bpw = rp // _NW
    c = bpw
    while c * (d + 1) * 4 > 400_000:
        c //= 2
    nchunks = bpw // c
    mesh = plsc.VectorSubcoreMesh(core_axis_name="c", subcore_axis_name="s")

    @functools.partial(
        pl.kernel,
        mesh=mesh,
        compiler_params=pltpu.CompilerParams(use_tc_tiling_on_sc=False),
        out_type=jax.ShapeDtypeStruct((rp, d), jnp.float32),
        scratch_types=[
            pltpu.VMEM((c,), jnp.int32),
            pltpu.VMEM((c, d), jnp.float32),
            pltpu.SemaphoreType.DMA,
        ],
    )
    def gk(table_hbm, idx_hbm, out_hbm, idx_v, rows_v, sem):
        wid = lax.axis_index("s") * 2 + lax.axis_index("c")
        base = wid * bpw
        for t in range(nchunks):
            off = base + t * c
            pltpu.sync_copy(idx_hbm.at[pl.ds(off, c)], idx_v)
            pltpu.async_copy(table_hbm.at[idx_v], rows_v, sem).wait()
            pltpu.sync_copy(rows_v, out_hbm.at[pl.ds(off, c)])

    out = gk(table, idx)
    return out[:r] if rp != r else out


def _mlp_pool(g, center, pos_off, extra, ws):
    k, r, C = g.shape
    ce = 0 if extra is None else extra.shape[1]
    cl = ws[-1].shape[0] if ws else C
    cap = max(8, min(512, 4_000_000 // (k * C * 4)))
    ts = _pick_ts(r, cap)
    nc = 0 if center is None else 1
    ne = 0 if extra is None else 1

    def kern(*refs):
        g_ref = refs[0]
        c_blk = refs[1][...] if nc else None
        e_blk = refs[1 + nc][...] if ne else None
        w_refs = refs[1 + nc + ne:-1]
        o_ref = refs[-1]
        sub = None
        if c_blk is not None:
            parts = []
            if pos_off:
                parts.append(jnp.zeros((ts, pos_off), jnp.float32))
            parts.append(c_blk)
            if C - pos_off - 16:
                parts.append(jnp.zeros((ts, C - pos_off - 16), jnp.float32))
            sub = parts[0] if len(parts) == 1 else jnp.concatenate(parts, 1)
        acc = None
        for kk in range(k):
            x = g_ref[kk]
            if sub is not None:
                x = x - sub
            if e_blk is not None:
                x = jnp.concatenate([x, e_blk], axis=1)
            h = x
            for wr in w_refs:
                h = jnp.maximum(_dotg(h, wr[...]) * _BN, 0.0)
            acc = h if acc is None else jnp.maximum(acc, h)
        o_ref[...] = acc

    in_specs = [pl.BlockSpec((k, ts, C), lambda i: (0, i, 0))]
    args = [g]
    if center is not None:
        in_specs.append(pl.BlockSpec((ts, 16), lambda i: (i, 0)))
        args.append(center)
    if extra is not None:
        in_specs.append(pl.BlockSpec((ts, ce), lambda i: (i, 0)))
        args.append(extra)
    for w in ws:
        in_specs.append(pl.BlockSpec(w.shape, lambda i: (0, 0)))
        args.append(w)
    return pl.pallas_call(
        kern,
        grid=(r // ts,),
        in_specs=in_specs,
        out_specs=pl.BlockSpec((ts, cl), lambda i: (i, 0)),
        out_shape=jax.ShapeDtypeStruct((r, cl), jnp.float32),
    )(*args)


def _linear_rows(xs, w0s, relus, more_ws=(), bias=None):
    r = xs[0].shape[0]
    cl = more_ws[-1].shape[0] if more_ws else w0s[0].shape[0]
    ts = _pick_ts(r, 512)
    n0 = len(xs)
    nm = len(more_ws)

    def kern(*refs):
        x_refs = refs[:n0]
        w0_refs = refs[n0:2 * n0]
        m_refs = refs[2 * n0:2 * n0 + nm]
        b_ref = refs[2 * n0 + nm] if bias is not None else None
        o_ref = refs[-1]
        acc = _dotg(x_refs[0][...], w0_refs[0][...])
        for xr, wr in zip(x_refs[1:], w0_refs[1:]):
            acc = acc + _dotg(xr[...], wr[...])
        if relus[0]:
            acc = jnp.maximum(acc * _BN, 0.0)
        for wr, rl in zip(m_refs, relus[1:]):
            acc = _dotg(acc, wr[...])
            if rl:
                acc = jnp.maximum(acc * _BN, 0.0)
        if b_ref is not None:
            acc = acc + b_ref[...]
        o_ref[...] = acc

    in_specs = [pl.BlockSpec((ts, x.shape[1]), lambda i: (i, 0)) for x in xs]
    in_specs += [pl.BlockSpec(w.shape, lambda i: (0, 0)) for w in w0s]
    in_specs += [pl.BlockSpec(w.shape, lambda i: (0, 0)) for w in more_ws]
    args = list(xs) + list(w0s) + list(more_ws)
    if bias is not None:
        in_specs.append(pl.BlockSpec((1, cl), lambda i: (0, 0)))
        args.append(bias)
    return pl.pallas_call(
        kern,
        grid=(r // ts,),
        in_specs=in_specs,
        out_specs=pl.BlockSpec((ts, cl), lambda i: (i, 0)),
        out_shape=jax.ShapeDtypeStruct((r, cl), jnp.float32),
    )(*args)


def _fp_head(g3, d3, f1r, wfp, w1, wc1, wc2, b2):
    r = g3.shape[1]
    ts = _pick_ts(r, 512)

    def kern(g_ref, d_ref, f_ref, wf_ref, w1_ref, wc1_ref, wc2_ref, b_ref,
             o_ref):
        d = jnp.maximum(d_ref[...], 0.0)
        w = 1.0 / (d + 1e-8)
        w = w / jnp.sum(w, axis=1, keepdims=True)
        interp = (w[:, 0:1] * g_ref[0] + w[:, 1:2] * g_ref[1]
                  + w[:, 2:3] * g_ref[2])
        x = jnp.concatenate([interp, f_ref[...]], axis=1)
        h = jnp.maximum(_dotg(x, wf_ref[...]) * _BN, 0.0)
        h = jnp.maximum(_dotg(h, w1_ref[...]) * _BN, 0.0)
        h = jnp.maximum(_dotg(h, wc1_ref[...]) * _BN, 0.0)
        o_ref[...] = _dotg(h, wc2_ref[...]) + b_ref[...]

    return pl.pallas_call(
        kern,
        grid=(r // ts,),
        in_specs=[
            pl.BlockSpec((3, ts, g3.shape[2]), lambda i: (0, i, 0)),
            pl.BlockSpec((ts, 3), lambda i: (i, 0)),
            pl.BlockSpec((ts, 16), lambda i: (i, 0)),
            pl.BlockSpec(wfp.shape, lambda i: (0, 0)),
            pl.BlockSpec(w1.shape, lambda i: (0, 0)),
            pl.BlockSpec(wc1.shape, lambda i: (0, 0)),
            pl.BlockSpec(wc2.shape, lambda i: (0, 0)),
            pl.BlockSpec((1, 8), lambda i: (0, 0)),
        ],
        out_specs=pl.BlockSpec((ts, 8), lambda i: (i, 0)),
        out_shape=jax.ShapeDtypeStruct((r, 8), jnp.float32),
    )(g3, d3, f1r, wfp, w1, wc1, wc2, b2)


def _rows(x):
    b_, ch, n = x.shape
    return jnp.transpose(x, (0, 2, 1)).reshape(b_ * n, ch)


def _pad_cols(x, w):
    if x.shape[1] == w:
        return x
    return jnp.concatenate(
        [x, jnp.zeros((x.shape[0], w - x.shape[1]), x.dtype)], axis=1)


def _t(rows_x, b_, n):
    return jnp.transpose(rows_x.reshape(b_, n, 16), (0, 2, 1))


def _nm_flat(idx):
    return jnp.transpose(idx, (2, 0, 1)).reshape(-1)


def _sa(xr, xt, fr, b_, n, npoint, radius, k, p, names):
    w0, w1, w2 = (p[nm + '_w'] for nm in names)
    c1 = w0.shape[0]
    cf = fr.shape[1]
    nf = w0.shape[1] - 3
    w0p = jnp.zeros((c1, 16 + cf), jnp.float32)
    w0p = w0p.at[:, :3].set(w0[:, :3])
    w0p = w0p.at[:, 16:16 + nf].set(w0[:, 3:])
    fi = _fps(xt, npoint)
    nxr = _gather_rows(xr, fi)
    nxt = _t(nxr, b_, npoint)
    idx = _ball(nxr.reshape(b_, npoint, 16), xr.reshape(b_, n, 16), radius, k)
    tbl = jnp.concatenate([xr, fr], axis=1)
    g = _gather_rows(tbl, _nm_flat(idx)).reshape(k, b_ * npoint, 16 + cf)
    f_out = _mlp_pool(g, nxr, 0, None, [w0p, w1, w2])
    return nxr, nxt, f_out


def kernel(pc1, pc2, feature1, feature2, params):
    p = params
    b_, _, n0 = pc1.shape

    pc1r = _pad_cols(_rows(pc1), 16)
    pc2r = _pad_cols(_rows(pc2), 16)
    pc1t = _t(pc1r, b_, n0)
    pc2t = _t(pc2r, b_, n0)
    f1r = _pad_cols(_rows(feature1), 16)
    f2r = _pad_cols(_rows(feature2), 16)

    sa1 = ['sa1_0', 'sa1_1', 'sa1_2']
    sa2 = ['sa2_0', 'sa2_1', 'sa2_2']

    l1p1r, l1p1t, l1f1 = _sa(pc1r, pc1t, f1r, b_, n0, 1024, 0.004, 16, p, sa1)
    l2p1r, l2p1t, l2f1 = _sa(l1p1r, l1p1t, l1f1, b_, 1024, 256, 0.008, 16, p, sa2)
    l1p2r, l1p2t, l1f2 = _sa(pc2r, pc2t, f2r, b_, n0, 1024, 0.004, 16, p, sa1)
    l2p2r, l2p2t, l2f2 = _sa(l1p2r, l1p2t, l1f2, b_, 1024, 256, 0.008, 16, p, sa2)

    fe0, fe1, fe2 = p['fe_0_w'], p['fe_1_w'], p['fe_2_w']
    w0p = jnp.zeros((fe0.shape[0], 16 + 128 + 128), jnp.float32)
    w0p = w0p.at[:, :3].set(fe0[:, :3])
    w0p = w0p.at[:, 16:144].set(fe0[:, 3:131])
    w0p = w0p.at[:, 144:272].set(fe0[:, 131:259])
    idx, _ = _knn(l2p1r.reshape(b_, 256, 16), l2p2r.reshape(b_, 256, 16), 64)
    tbl = jnp.concatenate([l2p2r, l2f2], axis=1)
    g = _gather_rows(tbl, _nm_flat(idx)).reshape(64, b_ * 256, 144)
    l2fnew = _mlp_pool(g, l2p1r, 0, l2f1, [w0p, fe1, fe2])

    sa3 = ['sa3_0', 'sa3_1', 'sa3_2']
    sa4 = ['sa4_0', 'sa4_1', 'sa4_2']
    l3p1r, l3p1t, l3f1 = _sa(l2p1r, l2p1t, l2fnew, b_, 256, 64, 0.016, 8, p, sa3)
    l4p1r, l4p1t, l4f1 = _sa(l3p1r, l3p1t, l3f1, b_, 64, 16, 0.032, 8, p, sa4)

    idx, _ = _knn(l3p1r.reshape(b_, 64, 16), l4p1r.reshape(b_, 16, 16), 8)
    tbl = jnp.concatenate([l4f1, l4p1r], axis=1)
    g = _gather_rows(tbl, _nm_flat(idx)).reshape(8, b_ * 64, 528)
    mx = _mlp_pool(g, l3p1r, 512, None, [])
    m2a = p['su1_m2_0_w']
    wa = jnp.zeros((m2a.shape[0], 528), jnp.float32)
    wa = wa.at[:, :515].set(m2a[:, :515])
    l3fnew = _linear_rows(
        [mx, l3f1], [wa, m2a[:, 515:]], [True, True],
        more_ws=[p['su1_m2_1_w']])

    def _su(p1r, p2r, f1rows, f2rows, s1, m1, m2):
        w0 = p[m1[0] + '_w']
        cf2 = w0.shape[1] - 3
        w0p = jnp.zeros((w0.shape[0], cf2 + 16), jnp.float32)
        w0p = w0p.at[:, :cf2 + 3].set(w0)
        s2 = p2r.shape[0] // b_
        idx2, _ = _knn(p1r.reshape(b_, s1, 16), p2r.reshape(b_, s2, 16), 8)
        tbl2 = jnp.concatenate([f2rows, p2r], axis=1)
        gg = _gather_rows(tbl2, _nm_flat(idx2)).reshape(8, b_ * s1, cf2 + 16)
        m1out = _mlp_pool(gg, p1r, cf2, None,
                          [w0p, p[m1[1] + '_w'], p[m1[2] + '_w']])
        wm = p[m2 + '_w']
        c1 = m1out.shape[1]
        return _linear_rows(
            [m1out, f1rows], [wm[:, :c1], wm[:, c1:]], [True])

    l2f1cat = jnp.concatenate([l2f1, l2fnew], axis=1)
    l2fnew1 = _su(l2p1r, l3p1r, l2f1cat, l3fnew, 256,
                  ['su2_m1_0', 'su2_m1_1', 'su2_m1_2'], 'su2_m2_0')
    l1fnew1 = _su(l1p1r, l2p1r, l1f1, l2fnew1, 1024,
                  ['su3_m1_0', 'su3_m1_1', 'su3_m1_2'], 'su3_m2_0')

    fp0 = p['fp_0_w']
    wfp = jnp.zeros((fp0.shape[0], 256 + 16), jnp.float32)
    wfp = wfp.at[:, :259].set(fp0)
    idx, d3 = _knn(pc1r.reshape(b_, n0, 16), l1p1r.reshape(b_, 1024, 16), 3,
                   want_d=True)
    g3 = _gather_rows(l1fnew1, _nm_flat(idx)).reshape(3, b_ * n0, 256)
    wc2 = jnp.concatenate(
        [p['conv2_w'], jnp.zeros((5, 128), jnp.float32)], axis=0)
    b2 = jnp.concatenate([p['conv2_b'], jnp.zeros((5,))]).reshape(1, 8)
    sf_rows = _fp_head(g3, d3.reshape(b_ * n0, 3), f1r, wfp,
                       p['fp_1_w'], p['conv1_w'], wc2, b2)
    sf = jnp.transpose(sf_rows[:, :3].reshape(b_, n0, 3), (0, 2, 1))
    return sf


# speedup vs baseline: 15.0803x; 2.8183x over previous
import functools

import jax
import jax.numpy as jnp
from jax import lax
from jax.experimental import pallas as pl
from jax.experimental.pallas import tpu as pltpu
from jax.experimental.pallas import tpu_sc as plsc

_BN = 1.0 / (1.0 + 1e-5) ** 0.5
_NW = 32


def _dotg(a, b_oc):
    return lax.dot_general(a, b_oc, (((1,), (1,)), ((), ())),
                           preferred_element_type=jnp.float32)


def _pick_ts(r, cap=1024):
    for t in (cap, 512, 256, 128, 64, 32, 16, 8):
        if t <= cap and r % t == 0 and t <= r:
            return t
    return r


def _fps_kernel(x_ref, o_ref, *, npoint, n, b_):
    x0, x1, x2 = x_ref[0], x_ref[1], x_ref[2]
    lane = lax.broadcasted_iota(jnp.int32, (b_, n), 1)
    olane = lax.broadcasted_iota(jnp.int32, (b_, npoint), 1)

    def body(i, st):
        dist, far, out = st
        out = out + far * (olane == i).astype(jnp.int32)
        sel = lane == far
        c0 = jnp.sum(jnp.where(sel, x0, 0.0), axis=1, keepdims=True)
        c1 = jnp.sum(jnp.where(sel, x1, 0.0), axis=1, keepdims=True)
        c2 = jnp.sum(jnp.where(sel, x2, 0.0), axis=1, keepdims=True)
        d = ((x0 - c0) ** 2 + (x1 - c1) ** 2) + (x2 - c2) ** 2
        dist = jnp.minimum(dist, d)
        m = jnp.max(dist, axis=1, keepdims=True)
        far = jnp.min(jnp.where(dist == m, lane, n), axis=1, keepdims=True)
        return dist, far, out

    dist0 = jnp.full((b_, n), 1e10, jnp.float32) + 0.0 * x0
    far0 = lane[:, :1] * 0
    out0 = lax.broadcasted_iota(jnp.int32, (b_, npoint), 0) * n
    _, _, out = lax.fori_loop(0, npoint, body, (dist0, far0, out0))
    o_ref[...] = out


def _fps(xt, npoint):
    b_, _, n = xt.shape
    xp = jnp.transpose(xt[:, :3, :], (1, 0, 2))
    out = pl.pallas_call(
        functools.partial(_fps_kernel, npoint=npoint, n=n, b_=b_),
        in_specs=[pl.BlockSpec((3, b_, n), lambda: (0, 0, 0))],
        out_specs=pl.BlockSpec((b_, npoint), lambda: (0, 0)),
        out_shape=jax.ShapeDtypeStruct((b_, npoint), jnp.int32),
    )(xp)
    return out.reshape(b_ * npoint)


def _sqd(q, x):
    qq = jnp.sum(q * q, axis=1, keepdims=True)
    xx = jnp.sum(x * x, axis=1, keepdims=True)
    dot = lax.dot_general(q, x, (((1,), (1,)), ((), ())),
                          preferred_element_type=jnp.float32)
    return qq + jnp.transpose(xx) - 2.0 * dot


def _ball_kernel(q_ref, x_ref, o_ref, *, k, r2, n):
    b = pl.program_id(0)
    d = _sqd(q_ref[0], x_ref[0])
    lane = lax.broadcasted_iota(jnp.int32, d.shape, 1)
    midx = jnp.where(d <= r2, lane, n)
    cols = []
    for _ in range(k):
        cur = jnp.min(midx, axis=1, keepdims=True)
        cols.append(cur)
        midx = jnp.where(midx == cur, n, midx)
    out = jnp.concatenate(cols, axis=1)
    out = jnp.where(out == n, cols[0], out)
    out = jnp.minimum(out, n - 1)
    o_ref[0] = out + b * n


def _ball(q_rows, x_rows, radius, k):
    b_, s, _ = q_rows.shape
    n = x_rows.shape[1]
    ts = _pick_ts(s, 256)
    out = pl.pallas_call(
        functools.partial(_ball_kernel, k=k, r2=radius * radius, n=n),
        grid=(b_, s // ts),
        in_specs=[
            pl.BlockSpec((1, ts, 16), lambda i, j: (i, j, 0)),
            pl.BlockSpec((1, n, 16), lambda i, j: (i, 0, 0)),
        ],
        out_specs=pl.BlockSpec((1, ts, k), lambda i, j: (i, j, 0)),
        out_shape=jax.ShapeDtypeStruct((b_, s, k), jnp.int32),
    )(q_rows, x_rows)
    return out


def _knn_kernel(q_ref, x_ref, *o_refs, k, n, want_d):
    b = pl.program_id(0)
    d = _sqd(q_ref[0], x_ref[0])
    lane = lax.broadcasted_iota(jnp.int32, d.shape, 1)
    icols, dcols = [], []
    for _ in range(k):
        m = jnp.min(d, axis=1, keepdims=True)
        j = jnp.min(jnp.where(d == m, lane, n), axis=1, keepdims=True)
        icols.append(j)
        dcols.append(m)
        d = jnp.where(lane == j, jnp.inf, d)
    o_refs[0][0] = jnp.concatenate(icols, axis=1) + b * n
    if want_d:
        o_refs[1][0] = jnp.concatenate(dcols, axis=1)


def _knn(q_rows, x_rows, k, want_d=False):
    b_, s, _ = q_rows.shape
    n = x_rows.shape[1]
    ts = _pick_ts(s, 256)
    shapes = [jax.ShapeDtypeStruct((b_, s, k), jnp.int32)]
    ospecs = [pl.BlockSpec((1, ts, k), lambda i, j: (i, j, 0))]
    if want_d:
        shapes.append(jax.ShapeDtypeStruct((b_, s, k), jnp.float32))
        ospecs.append(pl.BlockSpec((1, ts, k), lambda i, j: (i, j, 0)))
    out = pl.pallas_call(
        functools.partial(_knn_kernel, k=k, n=n, want_d=want_d),
        grid=(b_, s // ts),
        in_specs=[
            pl.BlockSpec((1, ts, 16), lambda i, j: (i, j, 0)),
            pl.BlockSpec((1, n, 16), lambda i, j: (i, 0, 0)),
        ],
        out_specs=ospecs,
        out_shape=shapes,
    )(q_rows, x_rows)
    return out if want_d else (out[0], None)


def _gather_rows(table, idx):
    r, = idx.shape
    d = table.shape[1]
    rp = -(-r // 256) * 256
    if rp != r:
        idx = jnp.concatenate([idx, jnp.zeros((rp - r,), jnp.int32)])
    bpw = rp // _NW
    c = bpw
    while c * (d + 1) * 4 > 400_000:
        c //= 2
    nchunks = bpw // c
    mesh = plsc.VectorSubcoreMesh(core_axis_name="c", subcore_axis_name="s")

    @functools.partial(
        pl.kernel,
        mesh=mesh,
        compiler_params=pltpu.CompilerParams(use_tc_tiling_on_sc=False),
        out_type=jax.ShapeDtypeStruct((rp, d), jnp.float32),
        scratch_types=[
            pltpu.VMEM((c,), jnp.int32),
            pltpu.VMEM((c, d), jnp.float32),
            pltpu.SemaphoreType.DMA,
        ],
    )
    def gk(table_hbm, idx_hbm, out_hbm, idx_v, rows_v, sem):
        wid = lax.axis_index("s") * 2 + lax.axis_index("c")
        base = wid * bpw
        for t in range(nchunks):
            off = base + t * c
            pltpu.sync_copy(idx_hbm.at[pl.ds(off, c)], idx_v)
            pltpu.async_copy(table_hbm.at[idx_v], rows_v, sem).wait()
            pltpu.sync_copy(rows_v, out_hbm.at[pl.ds(off, c)])

    out = gk(table, idx)
    return out[:r] if rp != r else out


def _mlp_pool(g, center, pos_off, extra, ws):
    k, r, C = g.shape
    ce = 0 if extra is None else extra.shape[1]
    cl = ws[-1].shape[0] if ws else C
    cap = max(8, min(512, 4_000_000 // (k * C * 4)))
    ts = _pick_ts(r, cap)
    nc = 0 if center is None else 1
    ne = 0 if extra is None else 1

    def kern(*refs):
        g_ref = refs[0]
        c_blk = refs[1][...] if nc else None
        e_blk = refs[1 + nc][...] if ne else None
        w_refs = refs[1 + nc + ne:-1]
        o_ref = refs[-1]
        sub = None
        if c_blk is not None:
            parts = []
            if pos_off:
                parts.append(jnp.zeros((ts, pos_off), jnp.float32))
            parts.append(c_blk)
            if C - pos_off - 16:
                parts.append(jnp.zeros((ts, C - pos_off - 16), jnp.float32))
            sub = parts[0] if len(parts) == 1 else jnp.concatenate(parts, 1)
        acc = None
        for kk in range(k):
            x = g_ref[kk]
            if sub is not None:
                x = x - sub
            if e_blk is not None:
                x = jnp.concatenate([x, e_blk], axis=1)
            h = x
            for wr in w_refs:
                h = jnp.maximum(_dotg(h, wr[...]) * _BN, 0.0)
            acc = h if acc is None else jnp.maximum(acc, h)
        o_ref[...] = acc

    in_specs = [pl.BlockSpec((k, ts, C), lambda i: (0, i, 0))]
    args = [g]
    if center is not None:
        in_specs.append(pl.BlockSpec((ts, 16), lambda i: (i, 0)))
        args.append(center)
    if extra is not None:
        in_specs.append(pl.BlockSpec((ts, ce), lambda i: (i, 0)))
        args.append(extra)
    for w in ws:
        in_specs.append(pl.BlockSpec(w.shape, lambda i: (0, 0)))
        args.append(w)
    return pl.pallas_call(
        kern,
        grid=(r // ts,),
        in_specs=in_specs,
        out_specs=pl.BlockSpec((ts, cl), lambda i: (i, 0)),
        out_shape=jax.ShapeDtypeStruct((r, cl), jnp.float32),
    )(*args)


def _linear_rows(xs, w0s, relus, more_ws=(), bias=None):
    r = xs[0].shape[0]
    cl = more_ws[-1].shape[0] if more_ws else w0s[0].shape[0]
    ts = _pick_ts(r, 512)
    n0 = len(xs)
    nm = len(more_ws)

    def kern(*refs):
        x_refs = refs[:n0]
        w0_refs = refs[n0:2 * n0]
        m_refs = refs[2 * n0:2 * n0 + nm]
        b_ref = refs[2 * n0 + nm] if bias is not None else None
        o_ref = refs[-1]
        acc = _dotg(x_refs[0][...], w0_refs[0][...])
        for xr, wr in zip(x_refs[1:], w0_refs[1:]):
            acc = acc + _dotg(xr[...], wr[...])
        if relus[0]:
            acc = jnp.maximum(acc * _BN, 0.0)
        for wr, rl in zip(m_refs, relus[1:]):
            acc = _dotg(acc, wr[...])
            if rl:
                acc = jnp.maximum(acc * _BN, 0.0)
        if b_ref is not None:
            acc = acc + b_ref[...]
        o_ref[...] = acc

    in_specs = [pl.BlockSpec((ts, x.shape[1]), lambda i: (i, 0)) for x in xs]
    in_specs += [pl.BlockSpec(w.shape, lambda i: (0, 0)) for w in w0s]
    in_specs += [pl.BlockSpec(w.shape, lambda i: (0, 0)) for w in more_ws]
    args = list(xs) + list(w0s) + list(more_ws)
    if bias is not None:
        in_specs.append(pl.BlockSpec((1, cl), lambda i: (0, 0)))
        args.append(bias)
    return pl.pallas_call(
        kern,
        grid=(r // ts,),
        in_specs=in_specs,
        out_specs=pl.BlockSpec((ts, cl), lambda i: (i, 0)),
        out_shape=jax.ShapeDtypeStruct((r, cl), jnp.float32),
    )(*args)


def _fp_head(g3, d3, f1r, wfp, w1, wc1, wc2, b2):
    r = g3.shape[1]
    ts = _pick_ts(r, 512)

    def kern(g_ref, d_ref, f_ref, wf_ref, w1_ref, wc1_ref, wc2_ref, b_ref,
             o_ref):
        d = jnp.maximum(d_ref[...], 0.0)
        w = 1.0 / (d + 1e-8)
        w = w / jnp.sum(w, axis=1, keepdims=True)
        interp = (w[:, 0:1] * g_ref[0] + w[:, 1:2] * g_ref[1]
                  + w[:, 2:3] * g_ref[2])
        x = jnp.concatenate([interp, f_ref[...]], axis=1)
        h = jnp.maximum(_dotg(x, wf_ref[...]) * _BN, 0.0)
        h = jnp.maximum(_dotg(h, w1_ref[...]) * _BN, 0.0)
        h = jnp.maximum(_dotg(h, wc1_ref[...]) * _BN, 0.0)
        o_ref[...] = _dotg(h, wc2_ref[...]) + b_ref[...]

    return pl.pallas_call(
        kern,
        grid=(r // ts,),
        in_specs=[
            pl.BlockSpec((3, ts, g3.shape[2]), lambda i: (0, i, 0)),
            pl.BlockSpec((ts, 3), lambda i: (i, 0)),
            pl.BlockSpec((ts, 16), lambda i: (i, 0)),
            pl.BlockSpec(wfp.shape, lambda i: (0, 0)),
            pl.BlockSpec(w1.shape, lambda i: (0, 0)),
            pl.BlockSpec(wc1.shape, lambda i: (0, 0)),
            pl.BlockSpec(wc2.shape, lambda i: (0, 0)),
            pl.BlockSpec((1, 8), lambda i: (0, 0)),
        ],
        out_specs=pl.BlockSpec((ts, 8), lambda i: (i, 0)),
        out_shape=jax.ShapeDtypeStruct((r, 8), jnp.float32),
    )(g3, d3, f1r, wfp, w1, wc1, wc2, b2)


def _rows(x):
    b_, ch, n = x.shape
    return jnp.transpose(x, (0, 2, 1)).reshape(b_ * n, ch)


def _pad_cols(x, w):
    if x.shape[1] == w:
        return x
    return jnp.concatenate(
        [x, jnp.zeros((x.shape[0], w - x.shape[1]), x.dtype)], axis=1)


def _t(rows_x, b_, n):
    return jnp.transpose(rows_x.reshape(b_, n, 16), (0, 2, 1))


def _nm_flat(idx):
    return jnp.transpose(idx, (2, 0, 1)).reshape(-1)


def _sa(xr, xt, fr, b_, n, npoint, radius, k, p, names):
    w0, w1, w2 = (p[nm + '_w'] for nm in names)
    c1 = w0.shape[0]
    cf = fr.shape[1]
    nf = w0.shape[1] - 3
    w0p = jnp.zeros((c1, 16 + cf), jnp.float32)
    w0p = w0p.at[:, :3].set(w0[:, :3])
    w0p = w0p.at[:, 16:16 + nf].set(w0[:, 3:])
    fi = _fps(xt, npoint)
    nxr = _gather_rows(xr, fi)
    nxt = _t(nxr, b_, npoint)
    idx = _ball(nxr.reshape(b_, npoint, 16), xr.reshape(b_, n, 16), radius, k)
    tbl = jnp.concatenate([xr, fr], axis=1)
    g = _gather_rows(tbl, _nm_flat(idx)).reshape(k, b_ * npoint, 16 + cf)
    f_out = _mlp_pool(g, nxr, 0, None, [w0p, w1, w2])
    return nxr, nxt, f_out


def kernel(pc1, pc2, feature1, feature2, params):
    p = params
    b_, _, n0 = pc1.shape

    pc1r = _pad_cols(_rows(pc1), 16)
    pc2r = _pad_cols(_rows(pc2), 16)
    pc1t = _t(pc1r, b_, n0)
    pc2t = _t(pc2r, b_, n0)
    f1r = _pad_cols(_rows(feature1), 16)
    f2r = _pad_cols(_rows(feature2), 16)

    sa1 = ['sa1_0', 'sa1_1', 'sa1_2']
    sa2 = ['sa2_0', 'sa2_1', 'sa2_2']

    l1p1r, l1p1t, l1f1 = _sa(pc1r, pc1t, f1r, b_, n0, 1024, 0.004, 16, p, sa1)
    l2p1r, l2p1t, l2f1 = _sa(l1p1r, l1p1t, l1f1, b_, 1024, 256, 0.008, 16, p, sa2)
    l1p2r, l1p2t, l1f2 = _sa(pc2r, pc2t, f2r, b_, n0, 1024, 0.004, 16, p, sa1)
    l2p2r, l2p2t, l2f2 = _sa(l1p2r, l1p2t, l1f2, b_, 1024, 256, 0.008, 16, p, sa2)

    fe0, fe1, fe2 = p['fe_0_w'], p['fe_1_w'], p['fe_2_w']
    w0p = jnp.zeros((fe0.shape[0], 16 + 128 + 128), jnp.float32)
    w0p = w0p.at[:, :3].set(fe0[:, :3])
    w0p = w0p.at[:, 16:144].set(fe0[:, 3:131])
    w0p = w0p.at[:, 144:272].set(fe0[:, 131:259])
    idx, _ = _knn(l2p1r.reshape(b_, 256, 16), l2p2r.reshape(b_, 256, 16), 64)
    tbl = jnp.concatenate([l2p2r, l2f2], axis=1)
    g = _gather_rows(tbl, _nm_flat(idx)).reshape(64, b_ * 256, 144)
    l2fnew = _mlp_pool(g, l2p1r, 0, l2f1, [w0p, fe1, fe2])

    sa3 = ['sa3_0', 'sa3_1', 'sa3_2']
    sa4 = ['sa4_0', 'sa4_1', 'sa4_2']
    l3p1r, l3p1t, l3f1 = _sa(l2p1r, l2p1t, l2fnew, b_, 256, 64, 0.016, 8, p, sa3)
    l4p1r, l4p1t, l4f1 = _sa(l3p1r, l3p1t, l3f1, b_, 64, 16, 0.032, 8, p, sa4)

    idx, _ = _knn(l3p1r.reshape(b_, 64, 16), l4p1r.reshape(b_, 16, 16), 8)
    tbl = jnp.concatenate([l4f1, l4p1r], axis=1)
    g = _gather_rows(tbl, _nm_flat(idx)).reshape(8, b_ * 64, 528)
    mx = _mlp_pool(g, l3p1r, 512, None, [])
    m2a = p['su1_m2_0_w']
    wa = jnp.zeros((m2a.shape[0], 528), jnp.float32)
    wa = wa.at[:, :515].set(m2a[:, :515])
    l3fnew = _linear_rows(
        [mx, l3f1], [wa, m2a[:, 515:]], [True, True],
        more_ws=[p['su1_m2_1_w']])

    def _su(p1r, p2r, f1rows, f2rows, s1, m1, m2):
        w0 = p[m1[0] + '_w']
        cf2 = w0.shape[1] - 3
        w0p = jnp.zeros((w0.shape[0], cf2 + 16), jnp.float32)
        w0p = w0p.at[:, :cf2 + 3].set(w0)
        s2 = p2r.shape[0] // b_
        idx2, _ = _knn(p1r.reshape(b_, s1, 16), p2r.reshape(b_, s2, 16), 8)
        tbl2 = jnp.concatenate([f2rows, p2r], axis=1)
        gg = _gather_rows(tbl2, _nm_flat(idx2)).reshape(8, b_ * s1, cf2 + 16)
        m1out = _mlp_pool(gg, p1r, cf2, None,
                          [w0p, p[m1[1] + '_w'], p[m1[2] + '_w']])
        wm = p[m2 + '_w']
        c1 = m1out.shape[1]
        return _linear_rows(
            [m1out, f1rows], [wm[:, :c1], wm[:, c1:]], [True])

    l2f1cat = jnp.concatenate([l2f1, l2fnew], axis=1)
    l2fnew1 = _su(l2p1r, l3p1r, l2f1cat, l3fnew, 256,
                  ['su2_m1_0', 'su2_m1_1', 'su2_m1_2'], 'su2_m2_0')
    l1fnew1 = _su(l1p1r, l2p1r, l1f1, l2fnew1, 1024,
                  ['su3_m1_0', 'su3_m1_1', 'su3_m1_2'], 'su3_m2_0')

    fp0 = p['fp_0_w']
    wfp = jnp.zeros((fp0.shape[0], 256 + 16), jnp.float32)
    wfp = wfp.at[:, :259].set(fp0)
    idx, d3 = _knn(pc1r.reshape(b_, n0, 16), l1p1r.reshape(b_, 1024, 16), 3,
                   want_d=True)
    g3 = _gather_rows(l1fnew1, _nm_flat(idx)).reshape(3, b_ * n0, 256)
    wc2 = jnp.concatenate(
        [p['conv2_w'], jnp.zeros((5, 128), jnp.float32)], axis=0)
    b2 = jnp.concatenate([p['conv2_b'], jnp.zeros((5,))]).reshape(1, 8)
    sf_rows = _fp_head(g3, d3.reshape(b_ * n0, 3), f1r, wfp,
                       p['fp_1_w'], p['conv1_w'], wc2, b2)
    sf = jnp.transpose(sf_rows[:, :3].reshape(b_, n0, 3), (0, 2, 1))
    return sf


# speedup vs baseline: 15.6923x; 1.0406x over previous
import functools

import jax
import jax.numpy as jnp
from jax import lax
from jax.experimental import pallas as pl
from jax.experimental.pallas import tpu as pltpu
from jax.experimental.pallas import tpu_sc as plsc

_BN = 1.0 / (1.0 + 1e-5) ** 0.5
_NW = 32


def _dotg(a, b_oc):
    return lax.dot_general(a, b_oc, (((1,), (1,)), ((), ())),
                           preferred_element_type=jnp.float32)


def _pick_ts(r, cap=1024):
    for t in (cap, 512, 256, 128, 64, 32, 16, 8):
        if t <= cap and r % t == 0 and t <= r:
            return t
    return r


def _fps_kernel(x_ref, o_ref, *, npoint, n, b_):
    x0, x1, x2 = x_ref[0], x_ref[1], x_ref[2]
    lane = lax.broadcasted_iota(jnp.int32, (b_, n), 1)
    olane = lax.broadcasted_iota(jnp.int32, (b_, npoint), 1)

    def body(i, st):
        dist, far, out = st
        out = out + far * (olane == i).astype(jnp.int32)
        sel = lane == far
        c0 = jnp.sum(jnp.where(sel, x0, 0.0), axis=1, keepdims=True)
        c1 = jnp.sum(jnp.where(sel, x1, 0.0), axis=1, keepdims=True)
        c2 = jnp.sum(jnp.where(sel, x2, 0.0), axis=1, keepdims=True)
        d = ((x0 - c0) ** 2 + (x1 - c1) ** 2) + (x2 - c2) ** 2
        dist = jnp.minimum(dist, d)
        m = jnp.max(dist, axis=1, keepdims=True)
        far = jnp.min(jnp.where(dist == m, lane, n), axis=1, keepdims=True)
        return dist, far, out

    dist0 = jnp.full((b_, n), 1e10, jnp.float32) + 0.0 * x0
    far0 = lane[:, :1] * 0
    out0 = lax.broadcasted_iota(jnp.int32, (b_, npoint), 0) * n
    _, _, out = lax.fori_loop(0, npoint, body, (dist0, far0, out0))
    o_ref[...] = out


def _fps(xt, npoint):
    b_, _, n = xt.shape
    xp = jnp.transpose(xt[:, :3, :], (1, 0, 2))
    out = pl.pallas_call(
        functools.partial(_fps_kernel, npoint=npoint, n=n, b_=b_),
        in_specs=[pl.BlockSpec((3, b_, n), lambda: (0, 0, 0))],
        out_specs=pl.BlockSpec((b_, npoint), lambda: (0, 0)),
        out_shape=jax.ShapeDtypeStruct((b_, npoint), jnp.int32),
    )(xp)
    return out.reshape(b_ * npoint)


def _sqd(q, x):
    qq = jnp.sum(q * q, axis=1, keepdims=True)
    xx = jnp.sum(x * x, axis=1, keepdims=True)
    dot = lax.dot_general(q, x, (((1,), (1,)), ((), ())),
                          preferred_element_type=jnp.float32)
    return qq + jnp.transpose(xx) - 2.0 * dot


def _ball_kernel(q_ref, x_ref, o_ref, *, k, r2, n):
    b = pl.program_id(0)
    d = _sqd(q_ref[0], x_ref[0])
    lane = lax.broadcasted_iota(jnp.int32, d.shape, 1)
    midx = jnp.where(d <= r2, lane, n)
    cols = []
    for _ in range(k):
        cur = jnp.min(midx, axis=1, keepdims=True)
        cols.append(cur)
        midx = jnp.where(midx == cur, n, midx)
    out = jnp.concatenate(cols, axis=1)
    out = jnp.where(out == n, cols[0], out)
    out = jnp.minimum(out, n - 1)
    o_ref[0] = out + b * n


def _ball(q_rows, x_rows, radius, k):
    b_, s, _ = q_rows.shape
    n = x_rows.shape[1]
    ts = _pick_ts(s, 256)
    out = pl.pallas_call(
        functools.partial(_ball_kernel, k=k, r2=radius * radius, n=n),
        grid=(b_, s // ts),
        in_specs=[
            pl.BlockSpec((1, ts, 16), lambda i, j: (i, j, 0)),
            pl.BlockSpec((1, n, 16), lambda i, j: (i, 0, 0)),
        ],
        out_specs=pl.BlockSpec((1, ts, k), lambda i, j: (i, j, 0)),
        out_shape=jax.ShapeDtypeStruct((b_, s, k), jnp.int32),
    )(q_rows, x_rows)
    return out


def _knn_kernel(q_ref, x_ref, *o_refs, k, n, want_d):
    b = pl.program_id(0)
    d = _sqd(q_ref[0], x_ref[0])
    lane = lax.broadcasted_iota(jnp.int32, d.shape, 1)
    icols, dcols = [], []
    for _ in range(k):
        m = jnp.min(d, axis=1, keepdims=True)
        j = jnp.min(jnp.where(d == m, lane, n), axis=1, keepdims=True)
        icols.append(j)
        dcols.append(m)
        d = jnp.where(lane == j, jnp.inf, d)
    o_refs[0][0] = jnp.concatenate(icols, axis=1) + b * n
    if want_d:
        o_refs[1][0] = jnp.concatenate(dcols, axis=1)


def _knn(q_rows, x_rows, k, want_d=False):
    b_, s, _ = q_rows.shape
    n = x_rows.shape[1]
    ts = _pick_ts(s, 256)
    shapes = [jax.ShapeDtypeStruct((b_, s, k), jnp.int32)]
    ospecs = [pl.BlockSpec((1, ts, k), lambda i, j: (i, j, 0))]
    if want_d:
        shapes.append(jax.ShapeDtypeStruct((b_, s, k), jnp.float32))
        ospecs.append(pl.BlockSpec((1, ts, k), lambda i, j: (i, j, 0)))
    out = pl.pallas_call(
        functools.partial(_knn_kernel, k=k, n=n, want_d=want_d),
        grid=(b_, s // ts),
        in_specs=[
            pl.BlockSpec((1, ts, 16), lambda i, j: (i, j, 0)),
            pl.BlockSpec((1, n, 16), lambda i, j: (i, 0, 0)),
        ],
        out_specs=ospecs,
        out_shape=shapes,
    )(q_rows, x_rows)
    return out if want_d else (out[0], None)


def _gather_rows(table, idx):
    r, = idx.shape
    d = table.shape[1]
    rp = -(-r // 256) * 256
    if rp != r:
        idx = jnp.concatenate([idx, jnp.zeros((rp - r,), jnp.int32)])
    bpw = rp // _NW
    c = bpw
    while c * (d + 1) * 4 > 400_000:
        c //= 2
    nchunks = bpw // c
    mesh = plsc.VectorSubcoreMesh(core_axis_name="c", subcore_axis_name="s")

    @functools.partial(
        pl.kernel,
        mesh=mesh,
        compiler_params=pltpu.CompilerParams(use_tc_tiling_on_sc=False),
        out_type=jax.ShapeDtypeStruct((rp, d), jnp.float32),
        scratch_types=[
            pltpu.VMEM((c,), jnp.int32),
            pltpu.VMEM((c, d), jnp.float32),
            pltpu.SemaphoreType.DMA,
        ],
    )
    def gk(table_hbm, idx_hbm, out_hbm, idx_v, rows_v, sem):
        wid = lax.axis_index("s") * 2 + lax.axis_index("c")
        base = wid * bpw
        for t in range(nchunks):
            off = base + t * c
            pltpu.sync_copy(idx_hbm.at[pl.ds(off, c)], idx_v)
            pltpu.async_copy(table_hbm.at[idx_v], rows_v, sem).wait()
            pltpu.sync_copy(rows_v, out_hbm.at[pl.ds(off, c)])

    out = gk(table, idx)
    return out[:r] if rp != r else out


def _mlp_pool(g, center, pos_off, extra, ws):
    k, r, C = g.shape
    ce = 0 if extra is None else extra.shape[1]
    cl = ws[-1].shape[0] if ws else C
    cap = max(8, min(512, 4_000_000 // (k * C * 4)))
    ts = _pick_ts(r, cap)
    nc = 0 if center is None else 1
    ne = 0 if extra is None else 1

    def kern(*refs):
        g_ref = refs[0]
        c_blk = refs[1][...] if nc else None
        e_blk = refs[1 + nc][...] if ne else None
        w_refs = refs[1 + nc + ne:-1]
        o_ref = refs[-1]
        sub = None
        if c_blk is not None:
            cwc = c_blk.shape[1]
            parts = []
            if pos_off:
                parts.append(jnp.zeros((ts, pos_off), jnp.float32))
            parts.append(c_blk)
            if C - pos_off - cwc:
                parts.append(jnp.zeros((ts, C - pos_off - cwc), jnp.float32))
            sub = parts[0] if len(parts) == 1 else jnp.concatenate(parts, 1)
        acc = None
        for kk in range(k):
            x = g_ref[kk]
            if sub is not None:
                x = x - sub
            if e_blk is not None:
                x = jnp.concatenate([x, e_blk], axis=1)
            h = x
            for wr in w_refs:
                h = jnp.maximum(_dotg(h, wr[...]) * _BN, 0.0)
            acc = h if acc is None else jnp.maximum(acc, h)
        o_ref[...] = acc

    in_specs = [pl.BlockSpec((k, ts, C), lambda i: (0, i, 0))]
    args = [g]
    if center is not None:
        in_specs.append(pl.BlockSpec((ts, center.shape[1]), lambda i: (i, 0)))
        args.append(center)
    if extra is not None:
        in_specs.append(pl.BlockSpec((ts, ce), lambda i: (i, 0)))
        args.append(extra)
    for w in ws:
        in_specs.append(pl.BlockSpec(w.shape, lambda i: (0, 0)))
        args.append(w)
    return pl.pallas_call(
        kern,
        grid=(r // ts,),
        in_specs=in_specs,
        out_specs=pl.BlockSpec((ts, cl), lambda i: (i, 0)),
        out_shape=jax.ShapeDtypeStruct((r, cl), jnp.float32),
    )(*args)


def _linear_rows(xs, w0s, relus, more_ws=(), bias=None):
    r = xs[0].shape[0]
    cl = more_ws[-1].shape[0] if more_ws else w0s[0].shape[0]
    ts = _pick_ts(r, 512)
    n0 = len(xs)
    nm = len(more_ws)

    def kern(*refs):
        x_refs = refs[:n0]
        w0_refs = refs[n0:2 * n0]
        m_refs = refs[2 * n0:2 * n0 + nm]
        b_ref = refs[2 * n0 + nm] if bias is not None else None
        o_ref = refs[-1]
        acc = _dotg(x_refs[0][...], w0_refs[0][...])
        for xr, wr in zip(x_refs[1:], w0_refs[1:]):
            acc = acc + _dotg(xr[...], wr[...])
        if relus[0]:
            acc = jnp.maximum(acc * _BN, 0.0)
        for wr, rl in zip(m_refs, relus[1:]):
            acc = _dotg(acc, wr[...])
            if rl:
                acc = jnp.maximum(acc * _BN, 0.0)
        if b_ref is not None:
            acc = acc + b_ref[...]
        o_ref[...] = acc

    in_specs = [pl.BlockSpec((ts, x.shape[1]), lambda i: (i, 0)) for x in xs]
    in_specs += [pl.BlockSpec(w.shape, lambda i: (0, 0)) for w in w0s]
    in_specs += [pl.BlockSpec(w.shape, lambda i: (0, 0)) for w in more_ws]
    args = list(xs) + list(w0s) + list(more_ws)
    if bias is not None:
        in_specs.append(pl.BlockSpec((1, cl), lambda i: (0, 0)))
        args.append(bias)
    return pl.pallas_call(
        kern,
        grid=(r // ts,),
        in_specs=in_specs,
        out_specs=pl.BlockSpec((ts, cl), lambda i: (i, 0)),
        out_shape=jax.ShapeDtypeStruct((r, cl), jnp.float32),
    )(*args)


def _fp_head(g3, d3, f1r, wfp, w1, wc1, wc2, b2):
    r = g3.shape[1]
    ts = _pick_ts(r, 512)

    def kern(g_ref, d_ref, f_ref, wf_ref, w1_ref, wc1_ref, wc2_ref, b_ref,
             o_ref):
        d = jnp.maximum(d_ref[...], 0.0)
        w = 1.0 / (d + 1e-8)
        w = w / jnp.sum(w, axis=1, keepdims=True)
        interp = (w[:, 0:1] * g_ref[0] + w[:, 1:2] * g_ref[1]
                  + w[:, 2:3] * g_ref[2])
        x = jnp.concatenate([interp, f_ref[...]], axis=1)
        h = jnp.maximum(_dotg(x, wf_ref[...]) * _BN, 0.0)
        h = jnp.maximum(_dotg(h, w1_ref[...]) * _BN, 0.0)
        h = jnp.maximum(_dotg(h, wc1_ref[...]) * _BN, 0.0)
        o_ref[...] = _dotg(h, wc2_ref[...]) + b_ref[...]

    return pl.pallas_call(
        kern,
        grid=(r // ts,),
        in_specs=[
            pl.BlockSpec((3, ts, g3.shape[2]), lambda i: (0, i, 0)),
            pl.BlockSpec((ts, 3), lambda i: (i, 0)),
            pl.BlockSpec((ts, 16), lambda i: (i, 0)),
            pl.BlockSpec(wfp.shape, lambda i: (0, 0)),
            pl.BlockSpec(w1.shape, lambda i: (0, 0)),
            pl.BlockSpec(wc1.shape, lambda i: (0, 0)),
            pl.BlockSpec(wc2.shape, lambda i: (0, 0)),
            pl.BlockSpec((1, 8), lambda i: (0, 0)),
        ],
        out_specs=pl.BlockSpec((ts, 8), lambda i: (i, 0)),
        out_shape=jax.ShapeDtypeStruct((r, 8), jnp.float32),
    )(g3, d3, f1r, wfp, w1, wc1, wc2, b2)


def _rows(x):
    b_, ch, n = x.shape
    return jnp.transpose(x, (0, 2, 1)).reshape(b_ * n, ch)


def _pad_cols(x, w):
    if x.shape[1] == w:
        return x
    return jnp.concatenate(
        [x, jnp.zeros((x.shape[0], w - x.shape[1]), x.dtype)], axis=1)


def _t(rows_x, b_, n):
    return jnp.transpose(rows_x.reshape(b_, n, 16), (0, 2, 1))


def _nm_flat(idx):
    return jnp.transpose(idx, (2, 0, 1)).reshape(-1)


def _sa(xr, xt, fr, b_, n, npoint, radius, k, p, names):
    w0, w1, w2 = (p[nm + '_w'] for nm in names)
    c1 = w0.shape[0]
    nf = w0.shape[1] - 3
    cw = -(-(3 + nf) // 8) * 8
    w0p = jnp.zeros((c1, cw), jnp.float32)
    w0p = w0p.at[:, :3 + nf].set(w0)
    fi = _fps(xt, npoint)
    nxr = _gather_rows(xr, fi)
    nxt = _t(nxr, b_, npoint)
    idx = _ball(nxr.reshape(b_, npoint, 16), xr.reshape(b_, n, 16), radius, k)
    tbl = _pad_cols(jnp.concatenate([xr[:, :3], fr[:, :nf]], axis=1), cw)
    g = _gather_rows(tbl, _nm_flat(idx)).reshape(k, b_ * npoint, cw)
    f_out = _mlp_pool(g, nxr[:, :min(cw, 16)], 0, None, [w0p, w1, w2])
    return nxr, nxt, f_out


def kernel(pc1, pc2, feature1, feature2, params):
    p = params
    b_, _, n0 = pc1.shape

    pc1r = _pad_cols(_rows(pc1), 16)
    pc2r = _pad_cols(_rows(pc2), 16)
    pc1t = _t(pc1r, b_, n0)
    pc2t = _t(pc2r, b_, n0)
    f1r = _pad_cols(_rows(feature1), 16)
    f2r = _pad_cols(_rows(feature2), 16)

    sa1 = ['sa1_0', 'sa1_1', 'sa1_2']
    sa2 = ['sa2_0', 'sa2_1', 'sa2_2']

    l1p1r, l1p1t, l1f1 = _sa(pc1r, pc1t, f1r, b_, n0, 1024, 0.004, 16, p, sa1)
    l2p1r, l2p1t, l2f1 = _sa(l1p1r, l1p1t, l1f1, b_, 1024, 256, 0.008, 16, p, sa2)
    l1p2r, l1p2t, l1f2 = _sa(pc2r, pc2t, f2r, b_, n0, 1024, 0.004, 16, p, sa1)
    l2p2r, l2p2t, l2f2 = _sa(l1p2r, l1p2t, l1f2, b_, 1024, 256, 0.008, 16, p, sa2)

    fe0, fe1, fe2 = p['fe_0_w'], p['fe_1_w'], p['fe_2_w']
    w0p = jnp.zeros((fe0.shape[0], 16 + 128 + 128), jnp.float32)
    w0p = w0p.at[:, :3].set(fe0[:, :3])
    w0p = w0p.at[:, 16:144].set(fe0[:, 3:131])
    w0p = w0p.at[:, 144:272].set(fe0[:, 131:259])
    idx, _ = _knn(l2p1r.reshape(b_, 256, 16), l2p2r.reshape(b_, 256, 16), 64)
    tbl = jnp.concatenate([l2p2r, l2f2], axis=1)
    g = _gather_rows(tbl, _nm_flat(idx)).reshape(64, b_ * 256, 144)
    l2fnew = _mlp_pool(g, l2p1r, 0, l2f1, [w0p, fe1, fe2])

    sa3 = ['sa3_0', 'sa3_1', 'sa3_2']
    sa4 = ['sa4_0', 'sa4_1', 'sa4_2']
    l3p1r, l3p1t, l3f1 = _sa(l2p1r, l2p1t, l2fnew, b_, 256, 64, 0.016, 8, p, sa3)
    l4p1r, l4p1t, l4f1 = _sa(l3p1r, l3p1t, l3f1, b_, 64, 16, 0.032, 8, p, sa4)

    idx, _ = _knn(l3p1r.reshape(b_, 64, 16), l4p1r.reshape(b_, 16, 16), 8)
    tbl = jnp.concatenate([l4f1, l4p1r], axis=1)
    g = _gather_rows(tbl, _nm_flat(idx)).reshape(8, b_ * 64, 528)
    mx = _mlp_pool(g, l3p1r, 512, None, [])
    m2a = p['su1_m2_0_w']
    wa = jnp.zeros((m2a.shape[0], 528), jnp.float32)
    wa = wa.at[:, :515].set(m2a[:, :515])
    l3fnew = _linear_rows(
        [mx, l3f1], [wa, m2a[:, 515:]], [True, True],
        more_ws=[p['su1_m2_1_w']])

    def _su(p1r, p2r, f1rows, f2rows, s1, m1, m2):
        w0 = p[m1[0] + '_w']
        cf2 = w0.shape[1] - 3
        w0p = jnp.zeros((w0.shape[0], cf2 + 16), jnp.float32)
        w0p = w0p.at[:, :cf2 + 3].set(w0)
        s2 = p2r.shape[0] // b_
        idx2, _ = _knn(p1r.reshape(b_, s1, 16), p2r.reshape(b_, s2, 16), 8)
        tbl2 = jnp.concatenate([f2rows, p2r], axis=1)
        gg = _gather_rows(tbl2, _nm_flat(idx2)).reshape(8, b_ * s1, cf2 + 16)
        m1out = _mlp_pool(gg, p1r, cf2, None,
                          [w0p, p[m1[1] + '_w'], p[m1[2] + '_w']])
        wm = p[m2 + '_w']
        c1 = m1out.shape[1]
        return _linear_rows(
            [m1out, f1rows], [wm[:, :c1], wm[:, c1:]], [True])

    l2f1cat = jnp.concatenate([l2f1, l2fnew], axis=1)
    l2fnew1 = _su(l2p1r, l3p1r, l2f1cat, l3fnew, 256,
                  ['su2_m1_0', 'su2_m1_1', 'su2_m1_2'], 'su2_m2_0')
    l1fnew1 = _su(l1p1r, l2p1r, l1f1, l2fnew1, 1024,
                  ['su3_m1_0', 'su3_m1_1', 'su3_m1_2'], 'su3_m2_0')

    fp0 = p['fp_0_w']
    wfp = jnp.zeros((fp0.shape[0], 256 + 16), jnp.float32)
    wfp = wfp.at[:, :259].set(fp0)
    idx, d3 = _knn(pc1r.reshape(b_, n0, 16), l1p1r.reshape(b_, 1024, 16), 3,
                   want_d=True)
    g3 = _gather_rows(l1fnew1, _nm_flat(idx)).reshape(3, b_ * n0, 256)
    wc2 = jnp.concatenate(
        [p['conv2_w'], jnp.zeros((5, 128), jnp.float32)], axis=0)
    b2 = jnp.concatenate([p['conv2_b'], jnp.zeros((5,))]).reshape(1, 8)
    sf_rows = _fp_head(g3, d3.reshape(b_ * n0, 3), f1r, wfp,
                       p['fp_1_w'], p['conv1_w'], wc2, b2)
    sf = jnp.transpose(sf_rows[:, :3].reshape(b_, n0, 3), (0, 2, 1))
    return sf
